# baseline (speedup 1.0000x reference)
import jax
import jax.numpy as jnp
from jax import lax
from jax.experimental import pallas as pl
from jax.experimental.pallas import tpu as pltpu
from jax.experimental.pallas import tpu_sc as plsc

N = 10000
E = 320000
D = 128
H1 = 64
ZD = 32

NC = 2
NS = 16
NW = NC * NS
EPW = E // NW
RS = 632
RL = N - (NS - 1) * RS

CA = 2000
CB = 400
CG = 400
GPC = CG // 16

_f32 = jnp.float32
_i32 = jnp.int32

_SC_PARAMS = pltpu.CompilerParams(
    use_tc_tiling_on_sc=False, needs_layout_passes=False)


def _worker():
    cid = lax.axis_index("c")
    sid = lax.axis_index("s")
    return cid, sid, cid * NS + sid


def _striped(sid, fn):
    r0 = sid * RS

    @pl.when(sid != NS - 1)
    def _():
        fn(r0, RS)

    @pl.when(sid == NS - 1)
    def _():
        fn(r0, RL)


def _deg_body(src_h, dst_h, ones8, zeros8, dego, degi,
              sidx_all, didx_all, ones_v, acc_o, acc_i):
    cid, sid, wid = _worker()
    pltpu.sync_copy(ones8, ones_v)
    pltpu.sync_copy(src_h.at[wid], sidx_all)
    pltpu.sync_copy(dst_h.at[wid], didx_all)
    _striped(sid, lambda r0, nr: pltpu.sync_copy(
        zeros8.at[pl.ds(r0, nr)], acc_o.at[pl.ds(r0, nr)]))
    _striped(sid, lambda r0, nr: pltpu.sync_copy(
        zeros8.at[pl.ds(r0, nr)], acc_i.at[pl.ds(r0, nr)]))
    plsc.subcore_barrier()

    def chunk(k, carry):
        pltpu.sync_copy(ones_v, acc_o.at[sidx_all.at[k]], add=True)
        pltpu.sync_copy(ones_v, acc_i.at[didx_all.at[k]], add=True)
        return carry

    lax.fori_loop(0, EPW // CA, chunk, 0)
    plsc.subcore_barrier()
    _striped(sid, lambda r0, nr: pltpu.sync_copy(
        acc_o.at[pl.ds(r0, nr)], dego.at[cid, pl.ds(r0, nr)]))
    _striped(sid, lambda r0, nr: pltpu.sync_copy(
        acc_i.at[pl.ds(r0, nr)], degi.at[cid, pl.ds(r0, nr)]))


def _make_deg_call(mesh):
    return pl.kernel(
        _deg_body,
        out_type=[jax.ShapeDtypeStruct((NC, N, 8), _f32),
                  jax.ShapeDtypeStruct((NC, N, 8), _f32)],
        mesh=mesh,
        compiler_params=_SC_PARAMS,
        scratch_types=[
            pltpu.VMEM((EPW // CA, CA), _i32),
            pltpu.VMEM((EPW // CA, CA), _i32),
            pltpu.VMEM((CA, 8), _f32),
            pltpu.VMEM_SHARED((N, 8), _f32),
            pltpu.VMEM_SHARED((N, 8), _f32),
        ],
    )


def _seg_body(rows_h, src_h, dst_h, zrows, out_h,
              sidx_all, didx_all, rows0, rows1, acc, sem0, sem1):
    cid, sid, wid = _worker()
    NCH = EPW // CB
    bufs = ((rows0, sem0), (rows1, sem1))
    pltpu.sync_copy(src_h.at[wid], sidx_all)
    pltpu.sync_copy(dst_h.at[wid], didx_all)
    _striped(sid, lambda r0, nr: pltpu.sync_copy(
        zrows.at[pl.ds(r0, nr)], acc.at[pl.ds(r0, nr)]))
    plsc.subcore_barrier()

    def fire(j, b):
        rows_v, sem = bufs[b]
        pltpu.async_copy(rows_h.at[sidx_all.at[j]], rows_v, sem)

    def consume(j, b):
        rows_v, sem = bufs[b]
        pltpu.make_async_copy(rows_h.at[sidx_all.at[j]], rows_v, sem).wait()
        pltpu.sync_copy(rows_v, acc.at[didx_all.at[j]], add=True)

    fire(0, 0)
    fire(1, 1)

    def pair(m, carry):
        j0 = 2 * m
        consume(j0, 0)

        @pl.when(j0 + 2 < NCH)
        def _():
            fire(j0 + 2, 0)

        consume(j0 + 1, 1)

        @pl.when(j0 + 3 < NCH)
        def _():
            fire(j0 + 3, 1)

        return carry

    lax.fori_loop(0, (NCH - 1) // 2, pair, 0)
    consume(NCH - 1, 0)
    plsc.subcore_barrier()
    _striped(sid, lambda r0, nr: pltpu.sync_copy(
        acc.at[pl.ds(r0, nr)], out_h.at[cid, pl.ds(r0, nr)]))


def _make_seg_call(mesh):
    return pl.kernel(
        _seg_body,
        out_type=jax.ShapeDtypeStruct((NC, N, H1), _f32),
        mesh=mesh,
        compiler_params=_SC_PARAMS,
        scratch_types=[
            pltpu.VMEM((EPW // CB, CB), _i32),
            pltpu.VMEM((EPW // CB, CB), _i32),
            pltpu.VMEM((CB, H1), _f32),
            pltpu.VMEM((CB, H1), _f32),
            pltpu.VMEM_SHARED((N, H1), _f32),
            pltpu.SemaphoreType.DMA,
            pltpu.SemaphoreType.DMA,
        ],
    )


def _score_body(z_h, src_h, dst_h, negd, pos_h, neg_h,
                sidx_all, didx_all, nidx_all,
                zs0, zd0, zn0, zs1, zd1, zn1,
                pos_v, neg_v, sem0, sem1):
    cid, sid, wid = _worker()
    base = wid * EPW
    lane = lax.broadcasted_iota(_i32, (16,), 0)
    NCH = EPW // CG
    bufs = ((zs0, zd0, zn0, sem0), (zs1, zd1, zn1, sem1))
    pltpu.sync_copy(src_h.at[wid], sidx_all)
    pltpu.sync_copy(dst_h.at[wid], didx_all)
    pltpu.sync_copy(negd.at[wid], nidx_all)

    def fire(j, b):
        zs, zd, zn, sem = bufs[b]
        pltpu.async_copy(z_h.at[sidx_all.at[j]], zs, sem)
        pltpu.async_copy(z_h.at[didx_all.at[j]], zd, sem)
        pltpu.async_copy(z_h.at[nidx_all.at[j]], zn, sem)

    def drain(b):
        zs, zd, zn, sem = bufs[b]
        pltpu.make_async_copy(z_h.at[sidx_all.at[0]], zs, sem).wait()
        pltpu.make_async_copy(z_h.at[didx_all.at[0]], zd, sem).wait()
        pltpu.make_async_copy(z_h.at[nidx_all.at[0]], zn, sem).wait()

    def compute(j, b):
        zs_v, zd_v, zn_v, sem = bufs[b]
        e0 = base + j * CG

        @plsc.parallel_loop(0, GPC)
        def group(g):
            rows = g * 16 + lane
            ap = [jnp.zeros((16,), _f32) for _ in range(4)]
            an = [jnp.zeros((16,), _f32) for _ in range(4)]
            for d in range(ZD):
                col = jnp.bitwise_and(lane + d, ZD - 1)
                vs = plsc.load_gather(zs_v, [rows, col])
                vd = plsc.load_gather(zd_v, [rows, col])
                vn = plsc.load_gather(zn_v, [rows, col])
                ap[d % 4] = ap[d % 4] + vs * vd
                an[d % 4] = an[d % 4] + vs * vn
            pos_v[pl.ds(g * 16, 16)] = (ap[0] + ap[1]) + (ap[2] + ap[3])
            neg_v[pl.ds(g * 16, 16)] = (an[0] + an[1]) + (an[2] + an[3])

        pltpu.sync_copy(pos_v, pos_h.at[pl.ds(e0, CG)])
        pltpu.sync_copy(neg_v, neg_h.at[pl.ds(e0, CG)])

    fire(0, 0)
    fire(1, 1)

    def pair(m, carry):
        j0 = 2 * m
        drain(0)
        compute(j0, 0)

        @pl.when(j0 + 2 < NCH)
        def _():
            fire(j0 + 2, 0)

        drain(1)
        compute(j0 + 1, 1)

        @pl.when(j0 + 3 < NCH)
        def _():
            fire(j0 + 3, 1)

        return carry

    lax.fori_loop(0, (NCH - 1) // 2, pair, 0)
    drain(0)
    compute(NCH - 1, 0)


def _make_score_call(mesh):
    return pl.kernel(
        _score_body,
        out_type=[jax.ShapeDtypeStruct((E,), _f32),
                  jax.ShapeDtypeStruct((E,), _f32)],
        mesh=mesh,
        compiler_params=_SC_PARAMS,
        scratch_types=(
            [pltpu.VMEM((EPW // CG, CG), _i32)] * 3
            + [pltpu.VMEM((CG, ZD), _f32)] * 6
            + [pltpu.VMEM((CG,), _f32)] * 2
            + [pltpu.SemaphoreType.DMA, pltpu.SemaphoreType.DMA]
        ),
    )


_sc_calls_cache = {}


def _sc_calls():
    if "v" not in _sc_calls_cache:
        mesh = plsc.VectorSubcoreMesh(
            core_axis_name="c", subcore_axis_name="s",
            num_cores=NC, num_subcores=NS)
        _sc_calls_cache["v"] = (
            _make_deg_call(mesh), _make_seg_call(mesh), _make_score_call(mesh))
    return _sc_calls_cache["v"]


_HI = lax.Precision.HIGHEST
BN = 2000
GN = N // BN


def _tc_b0_body(x_ref, w1_ref, y1_ref):
    y1_ref[...] = jnp.dot(x_ref[...], w1_ref[...],
                          preferred_element_type=_f32, precision=_HI)


_tc_b0 = pl.pallas_call(
    _tc_b0_body,
    grid=(GN,),
    in_specs=[
        pl.BlockSpec((BN, D), lambda i: (i, 0)),
        pl.BlockSpec((D, H1), lambda i: (0, 0)),
    ],
    out_specs=pl.BlockSpec((BN, H1), lambda i: (i, 0)),
    out_shape=jax.ShapeDtypeStruct((N, H1), _f32),
)


def _tc_b_body(y1_ref, dego_ref, degi_ref, y1s_ref, diso_ref, disi_ref):
    dego = (dego_ref[0] + dego_ref[1])[:, 0:1]
    degi = (degi_ref[0] + degi_ref[1])[:, 0:1]
    diso = lax.rsqrt(jnp.maximum(dego, 1.0))
    disi = lax.rsqrt(jnp.maximum(degi, 1.0))
    y1s_ref[...] = y1_ref[...] * diso
    diso_ref[...] = diso
    disi_ref[...] = disi


_tc_b = pl.pallas_call(
    _tc_b_body,
    grid=(GN,),
    in_specs=[
        pl.BlockSpec((BN, H1), lambda i: (i, 0)),
        pl.BlockSpec((NC, BN, 8), lambda i: (0, i, 0)),
        pl.BlockSpec((NC, BN, 8), lambda i: (0, i, 0)),
    ],
    out_specs=[
        pl.BlockSpec((BN, H1), lambda i: (i, 0)),
        pl.BlockSpec((BN, 1), lambda i: (i, 0)),
        pl.BlockSpec((BN, 1), lambda i: (i, 0)),
    ],
    out_shape=[jax.ShapeDtypeStruct((N, H1), _f32),
               jax.ShapeDtypeStruct((N, 1), _f32),
               jax.ShapeDtypeStruct((N, 1), _f32)],
)


def _tc_d_body(t1_ref, disi_ref, diso_ref, b1_ref, g_ref):
    t1 = t1_ref[0] + t1_ref[1]
    h = jnp.maximum(t1 * disi_ref[...] + b1_ref[...], 0.0)
    g_ref[...] = h * diso_ref[...]


_tc_d = pl.pallas_call(
    _tc_d_body,
    grid=(GN,),
    in_specs=[
        pl.BlockSpec((NC, BN, H1), lambda i: (0, i, 0)),
        pl.BlockSpec((BN, 1), lambda i: (i, 0)),
        pl.BlockSpec((BN, 1), lambda i: (i, 0)),
        pl.BlockSpec((1, H1), lambda i: (0, 0)),
    ],
    out_specs=pl.BlockSpec((BN, H1), lambda i: (i, 0)),
    out_shape=jax.ShapeDtypeStruct((N, H1), _f32),
)


def _tc_f_body(t2_ref, disi_ref, wm_ref, wv_ref, eps_ref, z_ref, kl_ref):
    i = pl.program_id(0)
    agg2 = (t2_ref[0] + t2_ref[1]) * disi_ref[...]
    zm = jnp.dot(agg2, wm_ref[...], preferred_element_type=_f32, precision=_HI)
    zl = jnp.dot(agg2, wv_ref[...], preferred_element_type=_f32, precision=_HI)
    zs = jnp.exp(zl)
    z_ref[...] = zm + zs * eps_ref[...]
    blk = jnp.sum(-zl + 0.5 * (zs * zs + zm * zm - 1.0)).reshape(1, 1)

    @pl.when(i == 0)
    def _():
        kl_ref[...] = jnp.zeros((1, 1), _f32)

    kl_ref[...] += blk


_tc_f = pl.pallas_call(
    _tc_f_body,
    grid=(GN,),
    in_specs=[
        pl.BlockSpec((NC, BN, H1), lambda i: (0, i, 0)),
        pl.BlockSpec((BN, 1), lambda i: (i, 0)),
        pl.BlockSpec((H1, ZD), lambda i: (0, 0)),
        pl.BlockSpec((H1, ZD), lambda i: (0, 0)),
        pl.BlockSpec((BN, ZD), lambda i: (i, 0)),
    ],
    out_specs=[
        pl.BlockSpec((BN, ZD), lambda i: (i, 0)),
        pl.BlockSpec((1, 1), lambda i: (0, 0)),
    ],
    out_shape=[jax.ShapeDtypeStruct((N, ZD), _f32),
               jax.ShapeDtypeStruct((1, 1), _f32)],
)


def _softplus(x):
    return jnp.maximum(x, 0.0) + jnp.log1p(jnp.exp(-jnp.abs(x)))


def _tc_h_body(pos_ref, neg_ref, kl_ref, out_ref):
    s = jnp.sum(_softplus(-pos_ref[...]) + _softplus(neg_ref[...]))
    out_ref[...] = s.reshape(1, 1) / E + (0.5 / N) * (kl_ref[...] / N)


_tc_h = pl.pallas_call(
    _tc_h_body,
    out_shape=jax.ShapeDtypeStruct((1, 1), _f32),
)


def kernel(features, edge_index, neg_dst, eps, W1, b1, Wm, Wv):
    ones8 = jnp.ones((CA, 8), _f32)
    zeros8 = jnp.zeros((N, 8), _f32)
    zrows = jnp.zeros((N, H1), _f32)
    _deg_call, _seg_call, _score_call = _sc_calls()
    src_a = edge_index[0].reshape(NW, EPW // CA, CA)
    dst_a = edge_index[1].reshape(NW, EPW // CA, CA)
    src_b = edge_index[0].reshape(NW, EPW // CB, CB)
    dst_b = edge_index[1].reshape(NW, EPW // CB, CB)
    neg_b = neg_dst.reshape(NW, EPW // CG, CG)

    y1 = _tc_b0(features, W1)
    dego_p, degi_p = _deg_call(src_a, dst_a, ones8, zeros8)
    y1s, diso, disi = _tc_b(y1, dego_p, degi_p)
    t1_p = _seg_call(y1s, src_b, dst_b, zrows)
    g = _tc_d(t1_p, disi, diso, b1.reshape(1, H1))
    t2_p = _seg_call(g, src_b, dst_b, zrows)
    z, klsum = _tc_f(t2_p, disi, Wm, Wv, eps)
    pos, neg = _score_call(z, src_b, dst_b, neg_b)
    loss = _tc_h(pos.reshape(E // 128, 128), neg.reshape(E // 128, 128), klsum)
    return loss[0, 0]


# speedup vs baseline: 1.0079x; 1.0079x over previous
import jax
import jax.numpy as jnp
from jax import lax
from jax.experimental import pallas as pl
from jax.experimental.pallas import tpu as pltpu
from jax.experimental.pallas import tpu_sc as plsc

N = 10000
E = 320000
D = 128
H1 = 64
ZD = 32

NC = 2
NS = 16
NW = NC * NS
EPW = E // NW
RS = 632
RL = N - (NS - 1) * RS

CA = 2000
CB = 400
CG = 400
GPC = CG // 16

_f32 = jnp.float32
_i32 = jnp.int32

_SC_PARAMS = pltpu.CompilerParams(
    use_tc_tiling_on_sc=False, needs_layout_passes=False)


def _worker():
    cid = lax.axis_index("c")
    sid = lax.axis_index("s")
    return cid, sid, cid * NS + sid


def _striped(sid, fn):
    r0 = sid * RS

    @pl.when(sid != NS - 1)
    def _():
        fn(r0, RS)

    @pl.when(sid == NS - 1)
    def _():
        fn(r0, RL)


def _deg_body(src_h, dst_h, ones8, zeros8, dego, degi,
              sidx_all, didx_all, ones_v, acc_o, acc_i):
    cid, sid, wid = _worker()
    pltpu.sync_copy(ones8, ones_v)
    pltpu.sync_copy(src_h.at[wid], sidx_all)
    pltpu.sync_copy(dst_h.at[wid], didx_all)
    _striped(sid, lambda r0, nr: pltpu.sync_copy(
        zeros8.at[pl.ds(r0, nr)], acc_o.at[pl.ds(r0, nr)]))
    _striped(sid, lambda r0, nr: pltpu.sync_copy(
        zeros8.at[pl.ds(r0, nr)], acc_i.at[pl.ds(r0, nr)]))
    plsc.subcore_barrier()

    def chunk(k, carry):
        pltpu.sync_copy(ones_v, acc_o.at[sidx_all.at[k]], add=True)
        pltpu.sync_copy(ones_v, acc_i.at[didx_all.at[k]], add=True)
        return carry

    lax.fori_loop(0, EPW // CA, chunk, 0)
    plsc.subcore_barrier()
    _striped(sid, lambda r0, nr: pltpu.sync_copy(
        acc_o.at[pl.ds(r0, nr)], dego.at[cid, pl.ds(r0, nr)]))
    _striped(sid, lambda r0, nr: pltpu.sync_copy(
        acc_i.at[pl.ds(r0, nr)], degi.at[cid, pl.ds(r0, nr)]))


def _make_deg_call(mesh):
    return pl.kernel(
        _deg_body,
        out_type=[jax.ShapeDtypeStruct((NC, N, 8), _f32),
                  jax.ShapeDtypeStruct((NC, N, 8), _f32)],
        mesh=mesh,
        compiler_params=_SC_PARAMS,
        scratch_types=[
            pltpu.VMEM((EPW // CA, CA), _i32),
            pltpu.VMEM((EPW // CA, CA), _i32),
            pltpu.VMEM((CA, 8), _f32),
            pltpu.VMEM_SHARED((N, 8), _f32),
            pltpu.VMEM_SHARED((N, 8), _f32),
        ],
    )


def _seg_body(rows_h, src_h, dst_h, zrows, out_h,
              sidx_all, didx_all, rows0, rows1, acc, sem0, sem1):
    cid, sid, wid = _worker()
    NCH = EPW // CB
    bufs = ((rows0, sem0), (rows1, sem1))
    pltpu.sync_copy(src_h.at[wid], sidx_all)
    pltpu.sync_copy(dst_h.at[wid], didx_all)
    _striped(sid, lambda r0, nr: pltpu.sync_copy(
        zrows.at[pl.ds(r0, nr)], acc.at[pl.ds(r0, nr)]))
    plsc.subcore_barrier()

    def fire(j, b):
        rows_v, sem = bufs[b]
        pltpu.async_copy(rows_h.at[sidx_all.at[j]], rows_v, sem)

    def consume(j, b):
        rows_v, sem = bufs[b]
        pltpu.make_async_copy(rows_h.at[sidx_all.at[j]], rows_v, sem).wait()
        pltpu.sync_copy(rows_v, acc.at[didx_all.at[j]], add=True)

    fire(0, 0)
    fire(1, 1)

    def pair(m, carry):
        j0 = 2 * m
        consume(j0, 0)

        @pl.when(j0 + 2 < NCH)
        def _():
            fire(j0 + 2, 0)

        consume(j0 + 1, 1)

        @pl.when(j0 + 3 < NCH)
        def _():
            fire(j0 + 3, 1)

        return carry

    lax.fori_loop(0, (NCH - 1) // 2, pair, 0)
    consume(NCH - 1, 0)
    plsc.subcore_barrier()
    _striped(sid, lambda r0, nr: pltpu.sync_copy(
        acc.at[pl.ds(r0, nr)], out_h.at[cid, pl.ds(r0, nr)]))


def _make_seg_call(mesh):
    return pl.kernel(
        _seg_body,
        out_type=jax.ShapeDtypeStruct((NC, N, H1), _f32),
        mesh=mesh,
        compiler_params=_SC_PARAMS,
        scratch_types=[
            pltpu.VMEM((EPW // CB, CB), _i32),
            pltpu.VMEM((EPW // CB, CB), _i32),
            pltpu.VMEM((CB, H1), _f32),
            pltpu.VMEM((CB, H1), _f32),
            pltpu.VMEM_SHARED((N, H1), _f32),
            pltpu.SemaphoreType.DMA,
            pltpu.SemaphoreType.DMA,
        ],
    )


def _score_body(z_h, src_h, dst_h, negd, pos_h, neg_h,
                sidx_all, didx_all, nidx_all,
                zs0, zd0, zn0, zs1, zd1, zn1,
                pos_v, neg_v, sem0, sem1):
    cid, sid, wid = _worker()
    base = wid * EPW
    lane = lax.broadcasted_iota(_i32, (16,), 0)
    NCH = EPW // CG
    bufs = ((zs0, zd0, zn0, sem0), (zs1, zd1, zn1, sem1))
    pltpu.sync_copy(src_h.at[wid], sidx_all)
    pltpu.sync_copy(dst_h.at[wid], didx_all)
    pltpu.sync_copy(negd.at[wid], nidx_all)

    def fire(j, b):
        zs, zd, zn, sem = bufs[b]
        pltpu.async_copy(z_h.at[sidx_all.at[j]], zs, sem)
        pltpu.async_copy(z_h.at[didx_all.at[j]], zd, sem)
        pltpu.async_copy(z_h.at[nidx_all.at[j]], zn, sem)

    def drain(b):
        zs, zd, zn, sem = bufs[b]
        pltpu.make_async_copy(z_h.at[sidx_all.at[0]], zs, sem).wait()
        pltpu.make_async_copy(z_h.at[didx_all.at[0]], zd, sem).wait()
        pltpu.make_async_copy(z_h.at[nidx_all.at[0]], zn, sem).wait()

    def compute(j, b):
        zs_v, zd_v, zn_v, sem = bufs[b]
        e0 = base + j * CG

        @plsc.parallel_loop(0, GPC)
        def group(g):
            rows = g * 16 + lane
            ap = [jnp.zeros((16,), _f32) for _ in range(4)]
            an = [jnp.zeros((16,), _f32) for _ in range(4)]
            for d in range(ZD):
                col = jnp.bitwise_and(lane + d, ZD - 1)
                vs = plsc.load_gather(zs_v, [rows, col])
                vd = plsc.load_gather(zd_v, [rows, col])
                vn = plsc.load_gather(zn_v, [rows, col])
                ap[d % 4] = ap[d % 4] + vs * vd
                an[d % 4] = an[d % 4] + vs * vn
            pos_v[pl.ds(g * 16, 16)] = (ap[0] + ap[1]) + (ap[2] + ap[3])
            neg_v[pl.ds(g * 16, 16)] = (an[0] + an[1]) + (an[2] + an[3])

        pltpu.sync_copy(pos_v, pos_h.at[pl.ds(e0, CG)])
        pltpu.sync_copy(neg_v, neg_h.at[pl.ds(e0, CG)])

    fire(0, 0)
    fire(1, 1)

    def pair(m, carry):
        j0 = 2 * m
        drain(0)
        compute(j0, 0)

        @pl.when(j0 + 2 < NCH)
        def _():
            fire(j0 + 2, 0)

        drain(1)
        compute(j0 + 1, 1)

        @pl.when(j0 + 3 < NCH)
        def _():
            fire(j0 + 3, 1)

        return carry

    lax.fori_loop(0, (NCH - 1) // 2, pair, 0)
    drain(0)
    compute(NCH - 1, 0)


def _make_score_call(mesh):
    return pl.kernel(
        _score_body,
        out_type=[jax.ShapeDtypeStruct((E,), _f32),
                  jax.ShapeDtypeStruct((E,), _f32)],
        mesh=mesh,
        compiler_params=_SC_PARAMS,
        scratch_types=(
            [pltpu.VMEM((EPW // CG, CG), _i32)] * 3
            + [pltpu.VMEM((CG, ZD), _f32)] * 6
            + [pltpu.VMEM((CG,), _f32)] * 2
            + [pltpu.SemaphoreType.DMA, pltpu.SemaphoreType.DMA]
        ),
    )


_sc_calls_cache = {}


def _sc_calls():
    if "v" not in _sc_calls_cache:
        mesh = plsc.VectorSubcoreMesh(
            core_axis_name="c", subcore_axis_name="s",
            num_cores=NC, num_subcores=NS)
        _sc_calls_cache["v"] = (
            _make_deg_call(mesh), _make_seg_call(mesh), _make_score_call(mesh))
    return _sc_calls_cache["v"]


_HI = lax.Precision.HIGHEST
BN = 2000
GN = N // BN


def _tc_b_body(x_ref, w1_ref, dego_ref, degi_ref, y1s_ref, diso_ref, disi_ref):
    dego = (dego_ref[0] + dego_ref[1])[:, 0:1]
    degi = (degi_ref[0] + degi_ref[1])[:, 0:1]
    diso = lax.rsqrt(jnp.maximum(dego, 1.0))
    disi = lax.rsqrt(jnp.maximum(degi, 1.0))
    y1 = jnp.dot(x_ref[...], w1_ref[...],
                 preferred_element_type=_f32, precision=_HI)
    y1s_ref[...] = y1 * diso
    diso_ref[...] = diso
    disi_ref[...] = disi


_tc_b = pl.pallas_call(
    _tc_b_body,
    grid=(GN,),
    in_specs=[
        pl.BlockSpec((BN, D), lambda i: (i, 0)),
        pl.BlockSpec((D, H1), lambda i: (0, 0)),
        pl.BlockSpec((NC, BN, 8), lambda i: (0, i, 0)),
        pl.BlockSpec((NC, BN, 8), lambda i: (0, i, 0)),
    ],
    out_specs=[
        pl.BlockSpec((BN, H1), lambda i: (i, 0)),
        pl.BlockSpec((BN, 1), lambda i: (i, 0)),
        pl.BlockSpec((BN, 1), lambda i: (i, 0)),
    ],
    out_shape=[jax.ShapeDtypeStruct((N, H1), _f32),
               jax.ShapeDtypeStruct((N, 1), _f32),
               jax.ShapeDtypeStruct((N, 1), _f32)],
)


def _tc_d_body(t1_ref, disi_ref, diso_ref, b1_ref, g_ref):
    t1 = t1_ref[0] + t1_ref[1]
    h = jnp.maximum(t1 * disi_ref[...] + b1_ref[...], 0.0)
    g_ref[...] = h * diso_ref[...]


_tc_d = pl.pallas_call(
    _tc_d_body,
    grid=(GN,),
    in_specs=[
        pl.BlockSpec((NC, BN, H1), lambda i: (0, i, 0)),
        pl.BlockSpec((BN, 1), lambda i: (i, 0)),
        pl.BlockSpec((BN, 1), lambda i: (i, 0)),
        pl.BlockSpec((1, H1), lambda i: (0, 0)),
    ],
    out_specs=pl.BlockSpec((BN, H1), lambda i: (i, 0)),
    out_shape=jax.ShapeDtypeStruct((N, H1), _f32),
)


def _tc_f_body(t2_ref, disi_ref, wm_ref, wv_ref, eps_ref, z_ref, kl_ref):
    i = pl.program_id(0)
    agg2 = (t2_ref[0] + t2_ref[1]) * disi_ref[...]
    zm = jnp.dot(agg2, wm_ref[...], preferred_element_type=_f32, precision=_HI)
    zl = jnp.dot(agg2, wv_ref[...], preferred_element_type=_f32, precision=_HI)
    zs = jnp.exp(zl)
    z_ref[...] = zm + zs * eps_ref[...]
    blk = jnp.sum(-zl + 0.5 * (zs * zs + zm * zm - 1.0)).reshape(1, 1)

    @pl.when(i == 0)
    def _():
        kl_ref[...] = jnp.zeros((1, 1), _f32)

    kl_ref[...] += blk


_tc_f = pl.pallas_call(
    _tc_f_body,
    grid=(GN,),
    in_specs=[
        pl.BlockSpec((NC, BN, H1), lambda i: (0, i, 0)),
        pl.BlockSpec((BN, 1), lambda i: (i, 0)),
        pl.BlockSpec((H1, ZD), lambda i: (0, 0)),
        pl.BlockSpec((H1, ZD), lambda i: (0, 0)),
        pl.BlockSpec((BN, ZD), lambda i: (i, 0)),
    ],
    out_specs=[
        pl.BlockSpec((BN, ZD), lambda i: (i, 0)),
        pl.BlockSpec((1, 1), lambda i: (0, 0)),
    ],
    out_shape=[jax.ShapeDtypeStruct((N, ZD), _f32),
               jax.ShapeDtypeStruct((1, 1), _f32)],
)


def _softplus(x):
    return jnp.maximum(x, 0.0) + jnp.log1p(jnp.exp(-jnp.abs(x)))


def _tc_h_body(pos_ref, neg_ref, kl_ref, out_ref):
    s = jnp.sum(_softplus(-pos_ref[...]) + _softplus(neg_ref[...]))
    out_ref[...] = s.reshape(1, 1) / E + (0.5 / N) * (kl_ref[...] / N)


_tc_h = pl.pallas_call(
    _tc_h_body,
    out_shape=jax.ShapeDtypeStruct((1, 1), _f32),
)


def kernel(features, edge_index, neg_dst, eps, W1, b1, Wm, Wv):
    ones8 = jnp.ones((CA, 8), _f32)
    zeros8 = jnp.zeros((N, 8), _f32)
    zrows = jnp.zeros((N, H1), _f32)
    _deg_call, _seg_call, _score_call = _sc_calls()
    src_a = edge_index[0].reshape(NW, EPW // CA, CA)
    dst_a = edge_index[1].reshape(NW, EPW // CA, CA)
    src_b = edge_index[0].reshape(NW, EPW // CB, CB)
    dst_b = edge_index[1].reshape(NW, EPW // CB, CB)
    neg_b = neg_dst.reshape(NW, EPW // CG, CG)

    dego_p, degi_p = _deg_call(src_a, dst_a, ones8, zeros8)
    y1s, diso, disi = _tc_b(features, W1, dego_p, degi_p)
    t1_p = _seg_call(y1s, src_b, dst_b, zrows)
    g = _tc_d(t1_p, disi, diso, b1.reshape(1, H1))
    t2_p = _seg_call(g, src_b, dst_b, zrows)
    z, klsum = _tc_f(t2_p, disi, Wm, Wv, eps)
    pos, neg = _score_call(z, src_b, dst_b, neg_b)
    loss = _tc_h(pos.reshape(E // 128, 128), neg.reshape(E // 128, 128), klsum)
    return loss[0, 0]
